# baseline (speedup 1.0000x reference)
import functools

import jax
import jax.numpy as jnp
from jax import lax
from jax.experimental import pallas as pl
from jax.experimental.pallas import tpu as pltpu
from jax.experimental.pallas import tpu_sc as plsc

_B = 16
_TW = 25
_NX = 2048
_NT = 250
_NNEI = 2
_TMIN, _TMAX = 0.0, 4.0
_N = _B * _NX
_EB = 2 * _NNEI * _NX - _NNEI * (_NNEI + 1)
_EPAD = 8192
_F = 2 * _TW + 3
_L = 16
_HALO = 16

_DCANDS = (-2, -1, 1, 2)


def _edge_body(x_hbm, out_hbm, xpad_v, src_v, dst_v):
    nc = 2
    wid = lax.axis_index("s") * nc + lax.axis_index("c")

    @pl.when(wid < _B)
    def _():
        b = wid
        sentinel = jnp.full((_L,), -1e9, jnp.float32)
        xpad_v[pl.ds(0, _L)] = sentinel
        xpad_v[pl.ds(_HALO + _NX, _L)] = sentinel
        pltpu.sync_copy(x_hbm.at[pl.ds(b * _NX, _NX)], xpad_v.at[pl.ds(_HALO, _NX)])

        iota = lax.iota(jnp.int32, _L)
        ones = jnp.full((_L,), 1, jnp.int32)
        zeros = jnp.zeros((_L,), jnp.int32)
        fone = jnp.full((_L,), 1.0, jnp.float32)
        fzero = jnp.zeros((_L,), jnp.float32)
        lane0 = jnp.where(iota == 0, fone, fzero)
        x01 = xpad_v[pl.ds(_HALO, _L)]
        x12 = xpad_v[pl.ds(_HALO + 1, _L)]
        dx = jnp.sum((x12 - x01) * lane0)
        radius = _NNEI * dx + dx * 0.1

        def body(it, carry):
            i0 = it * _L
            xi = xpad_v[pl.ds(_HALO + i0, _L)]
            base_i = b * _NX + i0 + iota
            masks = []
            cnt = jnp.zeros((_L,), jnp.int32)
            for d in _DCANDS:
                xj = xpad_v[pl.ds(_HALO + i0 + d, _L)]
                m = jnp.abs(xj - xi) <= radius
                masks.append(m)
                cnt = cnt + jnp.where(m, ones, zeros)
            incl = plsc.cumsum(cnt)
            pos_base = carry + incl - cnt
            off = jnp.zeros((_L,), jnp.int32)
            for d, m in zip(_DCANDS, masks):
                pos = pos_base + off
                plsc.store_scatter(src_v, [pos], base_i + d, mask=m)
                plsc.store_scatter(dst_v, [pos], base_i, mask=m)
                off = off + jnp.where(m, ones, zeros)
            return carry + jnp.sum(cnt)

        lax.fori_loop(0, _NX // _L, body, jnp.int32(0))
        pltpu.sync_copy(src_v, out_hbm.at[pl.ds((2 * b) * _EPAD, _EPAD)])
        pltpu.sync_copy(dst_v, out_hbm.at[pl.ds((2 * b + 1) * _EPAD, _EPAD)])


@jax.jit
def _edge_call(xr):
    mesh = plsc.VectorSubcoreMesh(core_axis_name="c", subcore_axis_name="s")
    fn = functools.partial(
        pl.kernel,
        mesh=mesh,
        out_type=jax.ShapeDtypeStruct((_B * 2 * _EPAD,), jnp.int32),
        scratch_types=[
            pltpu.VMEM((2 * _HALO + _NX,), jnp.float32),
            pltpu.VMEM((_EPAD,), jnp.int32),
            pltpu.VMEM((_EPAD,), jnp.int32),
        ],
        compiler_params=pltpu.CompilerParams(needs_layout_passes=False),
    )(_edge_body)
    return fn(xr)


_GB = 8
_GN = _GB * _NX


_NG = _B // _GB


def _feat_body(
    steps_ref, alpha_ref, data_hbm, labels_hbm, x_ref, out_ref, dbuf, lbuf, sems
):
    g = pl.program_id(0)

    def start(gi, slot):
        pltpu.make_async_copy(
            data_hbm.at[:, pl.ds(gi * _GB, _GB), :], dbuf.at[slot], sems.at[slot, 0]
        ).start()
        pltpu.make_async_copy(
            labels_hbm.at[:, pl.ds(gi * _GB, _GB), :], lbuf.at[slot], sems.at[slot, 1]
        ).start()

    @pl.when(g == 0)
    def _():
        start(0, 0)

    @pl.when(g + 1 < _NG)
    def _():
        start(g + 1, (g + 1) % 2)

    slot = g % 2
    pltpu.make_async_copy(
        data_hbm.at[:, pl.ds(g * _GB, _GB), :], dbuf.at[slot], sems.at[slot, 0]
    ).wait()
    pltpu.make_async_copy(
        labels_hbm.at[:, pl.ds(g * _GB, _GB), :], lbuf.at[slot], sems.at[slot, 1]
    ).wait()

    tscale = jnp.float32((_TMAX - _TMIN) / (_NT - 1))
    trow = jnp.concatenate(
        [
            jnp.full(
                (1, _NX),
                steps_ref[g * _GB + k].astype(jnp.float32) * tscale,
                jnp.float32,
            )
            for k in range(_GB)
        ],
        axis=1,
    )
    arow = jnp.concatenate(
        [jnp.full((1, _NX), alpha_ref[g * _GB + k], jnp.float32) for k in range(_GB)],
        axis=1,
    )
    u = dbuf[slot].reshape(_TW, _GN)
    y = lbuf[slot].reshape(_TW, _GN)
    xrow = x_ref[...].reshape(1, _GN)
    out_ref[...] = jnp.concatenate([u, y, trow, xrow, arow], axis=0)


@jax.jit
def _feat_call(steps, var_alpha, data_t, labels_t, x3):
    return pl.pallas_call(
        _feat_body,
        grid=(_NG,),
        in_specs=[
            pl.BlockSpec(memory_space=pltpu.SMEM),
            pl.BlockSpec(memory_space=pltpu.SMEM),
            pl.BlockSpec(memory_space=pltpu.MemorySpace.HBM),
            pl.BlockSpec(memory_space=pltpu.MemorySpace.HBM),
            pl.BlockSpec((1, _GB, _NX), lambda g: (0, g, 0)),
        ],
        out_specs=pl.BlockSpec((_F, _GN), lambda g: (0, g)),
        out_shape=jax.ShapeDtypeStruct((_F, _N), jnp.float32),
        scratch_shapes=[
            pltpu.VMEM((2, _TW, _GB, _NX), jnp.float32),
            pltpu.VMEM((2, _TW, _GB, _NX), jnp.float32),
            pltpu.SemaphoreType.DMA((2, 2)),
        ],
    )(steps, var_alpha, data_t, labels_t, x3)


def _depad_body(ebuf_ref, out_ref):
    for b in range(_B):
        src = ebuf_ref[pl.ds((2 * b) * _EPAD, _EB)]
        dst = ebuf_ref[pl.ds((2 * b + 1) * _EPAD, _EB)]
        out_ref[0:1, pl.ds(b * _EB, _EB)] = src.reshape(1, _EB)
        out_ref[1:2, pl.ds(b * _EB, _EB)] = dst.reshape(1, _EB)


@jax.jit
def _depad_call(ebuf):
    return pl.pallas_call(
        _depad_body,
        out_shape=jax.ShapeDtypeStruct((2, _B * _EB), jnp.int32),
    )(ebuf)


def kernel(data, labels, x, var_alpha, steps):
    data_t = jnp.transpose(data, (1, 0, 2))
    labels_t = jnp.transpose(labels, (1, 0, 2))
    ebuf = _edge_call(x.reshape(_B * _NX))
    featT = _feat_call(steps, var_alpha, data_t, labels_t, x.reshape(1, _B, _NX))
    edge_index = _depad_call(ebuf)
    return featT.T, edge_index


# speedup vs baseline: 1.0185x; 1.0185x over previous
import functools

import jax
import jax.numpy as jnp
from jax import lax
from jax.experimental import pallas as pl
from jax.experimental.pallas import tpu as pltpu
from jax.experimental.pallas import tpu_sc as plsc

_B = 16
_TW = 25
_NX = 2048
_NT = 250
_NNEI = 2
_TMIN, _TMAX = 0.0, 4.0
_N = _B * _NX
_EB = 2 * _NNEI * _NX - _NNEI * (_NNEI + 1)
_EPAD = 8192
_F = 2 * _TW + 3
_L = 16
_HALO = 16

_DCANDS = (-2, -1, 1, 2)


def _edge_body(x_hbm, out_hbm, xpad_v, src_v, dst_v):
    nc = 2
    wid = lax.axis_index("s") * nc + lax.axis_index("c")

    @pl.when(wid < _B)
    def _():
        b = wid
        sentinel = jnp.full((_L,), -1e9, jnp.float32)
        xpad_v[pl.ds(0, _L)] = sentinel
        xpad_v[pl.ds(_HALO + _NX, _L)] = sentinel
        pltpu.sync_copy(x_hbm.at[pl.ds(b * _NX, _NX)], xpad_v.at[pl.ds(_HALO, _NX)])

        iota = lax.iota(jnp.int32, _L)
        ones = jnp.full((_L,), 1, jnp.int32)
        zeros = jnp.zeros((_L,), jnp.int32)
        fone = jnp.full((_L,), 1.0, jnp.float32)
        fzero = jnp.zeros((_L,), jnp.float32)
        lane0 = jnp.where(iota == 0, fone, fzero)
        x01 = xpad_v[pl.ds(_HALO, _L)]
        x12 = xpad_v[pl.ds(_HALO + 1, _L)]
        dx = jnp.sum((x12 - x01) * lane0)
        radius = _NNEI * dx + dx * 0.1

        def body(it, carry):
            i0 = it * _L
            xi = xpad_v[pl.ds(_HALO + i0, _L)]
            base_i = b * _NX + i0 + iota
            masks = []
            cnt = jnp.zeros((_L,), jnp.int32)
            for d in _DCANDS:
                xj = xpad_v[pl.ds(_HALO + i0 + d, _L)]
                m = jnp.abs(xj - xi) <= radius
                masks.append(m)
                cnt = cnt + jnp.where(m, ones, zeros)
            incl = plsc.cumsum(cnt)
            pos_base = carry + incl - cnt
            off = jnp.zeros((_L,), jnp.int32)
            for d, m in zip(_DCANDS, masks):
                pos = pos_base + off
                plsc.store_scatter(src_v, [pos], base_i + d, mask=m)
                plsc.store_scatter(dst_v, [pos], base_i, mask=m)
                off = off + jnp.where(m, ones, zeros)
            return carry + jnp.sum(cnt)

        lax.fori_loop(0, _NX // _L, body, jnp.int32(0))
        pltpu.sync_copy(src_v, out_hbm.at[pl.ds((2 * b) * _EPAD, _EPAD)])
        pltpu.sync_copy(dst_v, out_hbm.at[pl.ds((2 * b + 1) * _EPAD, _EPAD)])


@jax.jit
def _edge_call(xr):
    mesh = plsc.VectorSubcoreMesh(core_axis_name="c", subcore_axis_name="s")
    fn = functools.partial(
        pl.kernel,
        mesh=mesh,
        out_type=jax.ShapeDtypeStruct((_B * 2 * _EPAD,), jnp.int32),
        scratch_types=[
            pltpu.VMEM((2 * _HALO + _NX,), jnp.float32),
            pltpu.VMEM((_EPAD,), jnp.int32),
            pltpu.VMEM((_EPAD,), jnp.int32),
        ],
        compiler_params=pltpu.CompilerParams(needs_layout_passes=False),
    )(_edge_body)
    return fn(xr)


_GB = 8
_GN = _GB * _NX


def _feat_body(steps_ref, alpha_ref, data_ref, labels_ref, x_ref, out_ref):
    g = pl.program_id(0)
    tscale = jnp.float32((_TMAX - _TMIN) / (_NT - 1))
    trow = jnp.concatenate(
        [
            jnp.full(
                (1, _NX),
                steps_ref[g * _GB + k].astype(jnp.float32) * tscale,
                jnp.float32,
            )
            for k in range(_GB)
        ],
        axis=1,
    )
    arow = jnp.concatenate(
        [jnp.full((1, _NX), alpha_ref[g * _GB + k], jnp.float32) for k in range(_GB)],
        axis=1,
    )
    u = data_ref[...].reshape(_TW, _GN)
    y = labels_ref[...].reshape(_TW, _GN)
    xrow = x_ref[...].reshape(1, _GN)
    out_ref[...] = jnp.concatenate([u, y, trow, xrow, arow], axis=0)


@jax.jit
def _feat_call(steps, var_alpha, data_t, labels_t, x3):
    return pl.pallas_call(
        _feat_body,
        grid=(_B // _GB,),
        in_specs=[
            pl.BlockSpec(memory_space=pltpu.SMEM),
            pl.BlockSpec(memory_space=pltpu.SMEM),
            pl.BlockSpec((_TW, _GB, _NX), lambda g: (0, g, 0)),
            pl.BlockSpec((_TW, _GB, _NX), lambda g: (0, g, 0)),
            pl.BlockSpec((1, _GB, _NX), lambda g: (0, g, 0)),
        ],
        out_specs=pl.BlockSpec((_F, _GN), lambda g: (0, g)),
        out_shape=jax.ShapeDtypeStruct((_F, _N), jnp.float32),
    )(steps, var_alpha, data_t, labels_t, x3)


def _depad_body(ebuf_ref, out_ref):
    for b in range(_B):
        src = ebuf_ref[pl.ds((2 * b) * _EPAD, _EB)]
        dst = ebuf_ref[pl.ds((2 * b + 1) * _EPAD, _EB)]
        out_ref[0:1, pl.ds(b * _EB, _EB)] = src.reshape(1, _EB)
        out_ref[1:2, pl.ds(b * _EB, _EB)] = dst.reshape(1, _EB)


@jax.jit
def _depad_call(ebuf):
    return pl.pallas_call(
        _depad_body,
        out_shape=jax.ShapeDtypeStruct((2, _B * _EB), jnp.int32),
    )(ebuf)


def kernel(data, labels, x, var_alpha, steps):
    data_t = jnp.transpose(data, (1, 0, 2))
    labels_t = jnp.transpose(labels, (1, 0, 2))
    ebuf = _edge_call(x.reshape(_B * _NX))
    featT = _feat_call(steps, var_alpha, data_t, labels_t, x.reshape(1, _B, _NX))
    edge_index = _depad_call(ebuf)
    return featT.T, edge_index
